# baseline (speedup 1.0000x reference)
import functools

import jax
import jax.numpy as jnp
from jax import lax
from jax.experimental import pallas as pl
from jax.experimental.pallas import tpu as pltpu
from jax.experimental.pallas import tpu_sc as plsc


def _argmin_body(x_ref, e_ref, idx_ref, em2_ref, e2_ref):
    i = pl.program_id(0)

    @pl.when(i == 0)
    def _():
        em = e_ref[...]
        em2_ref[...] = em * -2.0
        e2_ref[...] = jnp.sum(em * em, axis=0, keepdims=True)

    xb = x_ref[...]
    d = jnp.dot(xb, em2_ref[...], preferred_element_type=jnp.float32)
    d = d + jnp.sum(xb * xb, axis=1, keepdims=True)
    d = d + e2_ref[...]
    idx_ref[0, 0, :] = jnp.argmin(d, axis=1).astype(jnp.int32)


def _code_indices_tc(x, embeddings, block_tokens):
    n_tok, d_emb = x.shape
    k = embeddings.shape[1]
    grid = n_tok // block_tokens
    idx = pl.pallas_call(
        _argmin_body,
        grid=(grid,),
        in_specs=[
            pl.BlockSpec((block_tokens, d_emb), lambda i: (i, 0)),
            pl.BlockSpec((d_emb, k), lambda i: (0, 0)),
        ],
        out_specs=pl.BlockSpec((1, 1, block_tokens), lambda i: (i, 0, 0)),
        out_shape=jax.ShapeDtypeStruct((grid, 1, block_tokens), jnp.int32),
        scratch_shapes=[
            pltpu.VMEM((d_emb, k), jnp.float32),
            pltpu.VMEM((1, k), jnp.float32),
        ],
    )(x, embeddings)
    return idx.reshape(n_tok)


def _gather_sc(table, idx, n_tok, d_emb):
    info = plsc.get_sparse_core_info()
    nw = info.num_cores * info.num_subcores
    b_per_w = n_tok // nw
    chunk = min(b_per_w, 512)
    n_chunks = b_per_w // chunk
    n_bufs = min(n_chunks, 3)
    mesh = plsc.VectorSubcoreMesh(core_axis_name="c", subcore_axis_name="s")

    @functools.partial(
        pl.kernel,
        mesh=mesh,
        out_type=jax.ShapeDtypeStruct((n_tok, d_emb), jnp.float32),
        scratch_types=[
            pltpu.VMEM((b_per_w,), jnp.int32),
            pltpu.VMEM((n_bufs, chunk, d_emb), jnp.float32),
            [pltpu.SemaphoreType.DMA] * n_bufs,
            [pltpu.SemaphoreType.DMA] * n_bufs,
        ],
        compiler_params=pltpu.CompilerParams(use_tc_tiling_on_sc=False),
    )
    def gather_kernel(table_hbm, idx_hbm, out_hbm, idx_v, rows_v, gsems, wsems):
        wid = lax.axis_index("s") * info.num_cores + lax.axis_index("c")
        base = wid * b_per_w
        pltpu.sync_copy(idx_hbm.at[pl.ds(base, b_per_w)], idx_v)

        def start_gather(c):
            return pltpu.async_copy(
                table_hbm.at[idx_v.at[pl.ds(c * chunk, chunk)]],
                rows_v.at[c % n_bufs],
                gsems[c % n_bufs],
            )

        gd = [None] * n_chunks
        wd = [None] * n_chunks
        for c in range(n_bufs):
            gd[c] = start_gather(c)
        for c in range(n_chunks):
            gd[c].wait()
            wd[c] = pltpu.async_copy(
                rows_v.at[c % n_bufs],
                out_hbm.at[pl.ds(base + c * chunk, chunk)],
                wsems[c % n_bufs],
            )
            if c + n_bufs < n_chunks:
                wd[c].wait()
                gd[c + n_bufs] = start_gather(c + n_bufs)
        for c in range(max(0, n_chunks - n_bufs), n_chunks):
            wd[c].wait()

    return gather_kernel(table, idx)


def kernel(x, embeddings):
    n_tok, d_emb = x.shape
    idx = _code_indices_tc(x, embeddings, block_tokens=512)
    table = embeddings.T
    return _gather_sc(table, idx, n_tok, d_emb)


# speedup vs baseline: 1.0754x; 1.0754x over previous
import functools

import jax
import jax.numpy as jnp
from jax import lax
from jax.experimental import pallas as pl
from jax.experimental.pallas import tpu as pltpu
from jax.experimental.pallas import tpu_sc as plsc


def _argmin_body(xt_ref, e_ref, idx_ref, em2_ref, e2_ref):
    i = pl.program_id(0)

    @pl.when(i == 0)
    def _():
        em = e_ref[...]
        em2_ref[...] = em * -2.0
        e2_ref[...] = jnp.sum(em * em, axis=0, keepdims=True)

    xbt = xt_ref[...]
    d = jax.lax.dot_general(
        xbt, em2_ref[...],
        dimension_numbers=(((0,), (0,)), ((), ())),
        preferred_element_type=jnp.float32,
    )
    x2 = jnp.sum(xbt * xbt, axis=0)
    d = d + x2.reshape(x2.shape[0], 1)
    d = d + e2_ref[...]
    idx_ref[0, 0, :] = jnp.argmin(d, axis=1).astype(jnp.int32)


def _code_indices_tc(xt, embeddings, block_tokens):
    d_emb, n_tok = xt.shape
    k = embeddings.shape[1]
    grid = n_tok // block_tokens
    idx = pl.pallas_call(
        _argmin_body,
        grid=(grid,),
        in_specs=[
            pl.BlockSpec((d_emb, block_tokens), lambda i: (0, i)),
            pl.BlockSpec((d_emb, k), lambda i: (0, 0)),
        ],
        out_specs=pl.BlockSpec((1, 1, block_tokens), lambda i: (i, 0, 0)),
        out_shape=jax.ShapeDtypeStruct((grid, 1, block_tokens), jnp.int32),
        scratch_shapes=[
            pltpu.VMEM((d_emb, k), jnp.float32),
            pltpu.VMEM((1, k), jnp.float32),
        ],
    )(xt, embeddings)
    return idx.reshape(n_tok)


def _gather_sc(table, idx, n_tok, d_emb):
    info = plsc.get_sparse_core_info()
    nw = info.num_cores * info.num_subcores
    b_per_w = n_tok // nw
    chunk = min(b_per_w, 512)
    n_chunks = b_per_w // chunk
    n_bufs = min(n_chunks, 3)
    mesh = plsc.VectorSubcoreMesh(core_axis_name="c", subcore_axis_name="s")

    @functools.partial(
        pl.kernel,
        mesh=mesh,
        out_type=jax.ShapeDtypeStruct((n_tok, d_emb), jnp.float32),
        scratch_types=[
            pltpu.VMEM((b_per_w,), jnp.int32),
            pltpu.VMEM((n_bufs, chunk, d_emb), jnp.float32),
            [pltpu.SemaphoreType.DMA] * n_bufs,
            [pltpu.SemaphoreType.DMA] * n_bufs,
        ],
        compiler_params=pltpu.CompilerParams(use_tc_tiling_on_sc=False),
    )
    def gather_kernel(table_hbm, idx_hbm, out_hbm, idx_v, rows_v, gsems, wsems):
        wid = lax.axis_index("s") * info.num_cores + lax.axis_index("c")
        base = wid * b_per_w
        pltpu.sync_copy(idx_hbm.at[pl.ds(base, b_per_w)], idx_v)

        def start_gather(c):
            return pltpu.async_copy(
                table_hbm.at[idx_v.at[pl.ds(c * chunk, chunk)]],
                rows_v.at[c % n_bufs],
                gsems[c % n_bufs],
            )

        gd = [None] * n_chunks
        wd = [None] * n_chunks
        for c in range(n_bufs):
            gd[c] = start_gather(c)
        for c in range(n_chunks):
            gd[c].wait()
            wd[c] = pltpu.async_copy(
                rows_v.at[c % n_bufs],
                out_hbm.at[pl.ds(base + c * chunk, chunk)],
                wsems[c % n_bufs],
            )
            if c + n_bufs < n_chunks:
                wd[c].wait()
                gd[c + n_bufs] = start_gather(c + n_bufs)
        for c in range(max(0, n_chunks - n_bufs), n_chunks):
            wd[c].wait()

    return gather_kernel(table, idx)


def kernel(x, embeddings):
    n_tok, d_emb = x.shape
    idx = _code_indices_tc(x.T, embeddings, block_tokens=512)
    table = embeddings.T
    return _gather_sc(table, idx, n_tok, d_emb)


# speedup vs baseline: 1.5092x; 1.4034x over previous
import functools

import jax
import jax.numpy as jnp
from jax import lax
from jax.experimental import pallas as pl
from jax.experimental.pallas import tpu as pltpu
from jax.experimental.pallas import tpu_sc as plsc


def _argmin_body(xt_ref, e_ref, idx_ref, em2_ref, e2_ref):
    i = pl.program_id(0)

    @pl.when(i == 0)
    def _():
        em = e_ref[...]
        em2_ref[...] = em * -2.0
        e2_ref[...] = jnp.sum(em * em, axis=0, keepdims=True)

    xbt = xt_ref[...]
    d = jax.lax.dot_general(
        xbt, em2_ref[...],
        dimension_numbers=(((0,), (0,)), ((), ())),
        preferred_element_type=jnp.float32,
    )
    x2 = jnp.sum(xbt * xbt, axis=0)
    d = d + x2.reshape(x2.shape[0], 1)
    d = d + e2_ref[...]
    idx_ref[0, 0, :] = jnp.argmin(d, axis=1).astype(jnp.int32)


def _code_indices_tc(xt, embeddings, block_tokens):
    d_emb, n_tok = xt.shape
    k = embeddings.shape[1]
    grid = n_tok // block_tokens
    idx = pl.pallas_call(
        _argmin_body,
        grid=(grid,),
        in_specs=[
            pl.BlockSpec((d_emb, block_tokens), lambda i: (0, i)),
            pl.BlockSpec((d_emb, k), lambda i: (0, 0)),
        ],
        out_specs=pl.BlockSpec((1, 1, block_tokens), lambda i: (i, 0, 0)),
        out_shape=jax.ShapeDtypeStruct((grid, 1, block_tokens), jnp.int32),
        scratch_shapes=[
            pltpu.VMEM((d_emb, k), jnp.float32),
            pltpu.VMEM((1, k), jnp.float32),
        ],
    )(xt, embeddings)
    return idx.reshape(n_tok)


def _gather_sc(table, idx, n_tok, d_emb):
    info = plsc.get_sparse_core_info()
    nw = info.num_cores * info.num_subcores
    b_per_w = n_tok // nw
    chunk = min(b_per_w, 512)
    n_chunks = b_per_w // chunk
    mesh = plsc.VectorSubcoreMesh(core_axis_name="c", subcore_axis_name="s")

    @functools.partial(
        pl.kernel,
        mesh=mesh,
        out_type=jax.ShapeDtypeStruct((n_tok, d_emb), jnp.float32),
        scratch_types=[
            pltpu.VMEM((b_per_w,), jnp.int32),
            pltpu.VMEM((chunk, d_emb), jnp.float32),
            pltpu.SemaphoreType.DMA,
        ],
        compiler_params=pltpu.CompilerParams(use_tc_tiling_on_sc=False),
    )
    def gather_kernel(table_hbm, idx_hbm, out_hbm, idx_v, rows_v, sem):
        wid = lax.axis_index("s") * info.num_cores + lax.axis_index("c")
        base = wid * b_per_w
        pltpu.sync_copy(idx_hbm.at[pl.ds(base, b_per_w)], idx_v)
        for c in range(n_chunks):
            pltpu.async_copy(
                table_hbm.at[idx_v.at[pl.ds(c * chunk, chunk)]],
                rows_v, sem,
            ).wait()
            pltpu.sync_copy(rows_v, out_hbm.at[pl.ds(base + c * chunk, chunk)])

    return gather_kernel(table, idx)


def kernel(x, embeddings):
    n_tok, d_emb = x.shape
    idx = _code_indices_tc(x.T, embeddings, block_tokens=512)
    table = embeddings.T
    return _gather_sc(table, idx, n_tok, d_emb)
